# Initial kernel scaffold; baseline (speedup 1.0000x reference)
import jax
import jax.numpy as jnp
from jax.experimental import pallas as pl


def kernel(ypred, hidden, W_fc, b_fc):
    raise NotImplementedError("write your pallas kernel here")


# speedup vs baseline: 18.3911x; 18.3911x over previous
import math

import jax
import jax.numpy as jnp
import numpy as np
from jax.experimental import pallas as pl

_N = 1024
_R = 6
_W = 8
_H = 64
_RMIN = 0.1
_RMAX = 10.0
_FC_IN = _R * _W * _H
_FC_OUT = 64
_LOG_RMAX_BY_RMIN = math.log(int(_RMAX / float(_RMIN)))

_TILE = 128


def _social_pool_kernel(x_col, y_col, x_row, y_row, hidden, w_fc, b_fc, out):
    xi = x_col[:, 0:1]
    yi = y_col[:, 0:1]
    xj = x_row[0:1, :]
    yj = y_row[0:1, :]

    x_diff = xj - xi
    y_diff = yj - yi
    d2 = x_diff * x_diff + y_diff * y_diff
    r = jnp.sqrt(d2)

    ring_f = jnp.where(
        r < _RMIN,
        jnp.full_like(r, -1.0),
        jnp.floor((_R - 1) * (jnp.log(r / _RMIN) / _LOG_RMAX_BY_RMIN)),
    )
    valid = (ring_f >= 0.0) & (ring_f < _R)
    ring_i = jnp.clip(ring_f, 0.0, _R - 1).astype(jnp.int32)

    theta = jnp.arctan2(y_diff, x_diff)
    wedge = (theta * _W / (2.0 * np.pi) + (_W // 2 - 1)).astype(jnp.int32)
    wedge_m = jnp.mod(wedge, _W)

    cell = ring_i * _W + wedge_m

    h = hidden[...]
    means = []
    for c in range(_R * _W):
        m = jnp.where(valid & (cell == c), 1.0, 0.0)
        s = jnp.dot(m, h, preferred_element_type=jnp.float32,
                    precision=jax.lax.Precision.HIGHEST)
        cnt = jnp.sum(m, axis=1, keepdims=True)
        means.append(jnp.where(cnt > 0.0, s / jnp.maximum(cnt, 1.0), 0.0))

    mean_flat = jnp.concatenate(means, axis=1)
    acc = jnp.dot(mean_flat, w_fc[...], preferred_element_type=jnp.float32,
                  precision=jax.lax.Precision.HIGHEST)
    out[...] = jnp.maximum(acc + b_fc[0:1, :], 0.0)


def kernel(ypred, hidden, W_fc, b_fc):
    yd = jax.lax.stop_gradient(ypred)
    x_col = yd[:, 0:1]
    y_col = yd[:, 1:2]
    x_row = yd[:, 0].reshape(1, _N)
    y_row = yd[:, 1].reshape(1, _N)
    b2 = b_fc.reshape(1, _FC_OUT)

    grid = (_N // _TILE,)
    return pl.pallas_call(
        _social_pool_kernel,
        grid=grid,
        in_specs=[
            pl.BlockSpec((_TILE, 1), lambda t: (t, 0)),
            pl.BlockSpec((_TILE, 1), lambda t: (t, 0)),
            pl.BlockSpec((1, _N), lambda t: (0, 0)),
            pl.BlockSpec((1, _N), lambda t: (0, 0)),
            pl.BlockSpec((_N, _H), lambda t: (0, 0)),
            pl.BlockSpec((_FC_IN, _FC_OUT), lambda t: (0, 0)),
            pl.BlockSpec((1, _FC_OUT), lambda t: (0, 0)),
        ],
        out_specs=pl.BlockSpec((_TILE, _FC_OUT), lambda t: (t, 0)),
        out_shape=jax.ShapeDtypeStruct((_N, _FC_OUT), jnp.float32),
    )(x_col, y_col, x_row, y_row, hidden, W_fc, b2)


# speedup vs baseline: 54.5470x; 2.9659x over previous
import math

import jax
import jax.numpy as jnp
import numpy as np
from jax.experimental import pallas as pl

_N = 1024
_R = 6
_W = 8
_H = 64
_RMIN = 0.1
_RMAX = 10.0
_FC_IN = _R * _W * _H
_FC_OUT = 64
_LOG_RMAX_BY_RMIN = math.log(int(_RMAX / float(_RMIN)))

_TILE = 128


def _social_pool_kernel(x_col, y_col, x_row, y_row, hidden, w_fc, b_fc, out):
    xi = x_col[:, 0:1]
    yi = y_col[:, 0:1]
    xj = x_row[0:1, :]
    yj = y_row[0:1, :]

    x_diff = xj - xi
    y_diff = yj - yi
    d2 = x_diff * x_diff + y_diff * y_diff
    r = jnp.sqrt(d2)

    ring_f = jnp.where(
        r < _RMIN,
        jnp.full_like(r, -1.0),
        jnp.floor((_R - 1) * (jnp.log(r / _RMIN) / _LOG_RMAX_BY_RMIN)),
    )
    valid = (ring_f >= 0.0) & (ring_f < _R)
    ring_i = jnp.clip(ring_f, 0.0, _R - 1).astype(jnp.int32)

    theta = jnp.arctan2(y_diff, x_diff)
    wedge = (theta * _W / (2.0 * np.pi) + (_W // 2 - 1)).astype(jnp.int32)
    wedge_m = jnp.mod(wedge, _W)

    cell = ring_i * _W + wedge_m

    h = hidden[...]
    means = []
    for c in range(_R * _W):
        m = jnp.where(valid & (cell == c), 1.0, 0.0).astype(jnp.bfloat16)
        s = jnp.dot(m, h, preferred_element_type=jnp.float32)
        cnt = s[:, _H:_H + 1]
        sc = s[:, :_H]
        means.append(jnp.where(cnt > 0.0, sc / jnp.maximum(cnt, 1.0), 0.0))

    mean_flat = jnp.concatenate(means, axis=1)
    acc = jnp.dot(mean_flat, w_fc[...], preferred_element_type=jnp.float32,
                  precision=jax.lax.Precision.HIGHEST)
    out[...] = jnp.maximum(acc + b_fc[0:1, :], 0.0)


def kernel(ypred, hidden, W_fc, b_fc):
    yd = jax.lax.stop_gradient(ypred)
    x_col = yd[:, 0:1]
    y_col = yd[:, 1:2]
    x_row = yd[:, 0].reshape(1, _N)
    y_row = yd[:, 1].reshape(1, _N)
    b2 = b_fc.reshape(1, _FC_OUT)
    hidden_aug = jnp.concatenate(
        [hidden, jnp.ones((_N, 1), jnp.float32)], axis=1
    ).astype(jnp.bfloat16)

    grid = (_N // _TILE,)
    return pl.pallas_call(
        _social_pool_kernel,
        grid=grid,
        in_specs=[
            pl.BlockSpec((_TILE, 1), lambda t: (t, 0)),
            pl.BlockSpec((_TILE, 1), lambda t: (t, 0)),
            pl.BlockSpec((1, _N), lambda t: (0, 0)),
            pl.BlockSpec((1, _N), lambda t: (0, 0)),
            pl.BlockSpec((_N, _H + 1), lambda t: (0, 0)),
            pl.BlockSpec((_FC_IN, _FC_OUT), lambda t: (0, 0)),
            pl.BlockSpec((1, _FC_OUT), lambda t: (0, 0)),
        ],
        out_specs=pl.BlockSpec((_TILE, _FC_OUT), lambda t: (t, 0)),
        out_shape=jax.ShapeDtypeStruct((_N, _FC_OUT), jnp.float32),
    )(x_col, y_col, x_row, y_row, hidden_aug, W_fc, b2)


# speedup vs baseline: 66.0366x; 1.2106x over previous
import math

import jax
import jax.numpy as jnp
import numpy as np
from jax.experimental import pallas as pl

_N = 1024
_R = 6
_W = 8
_H = 64
_RMIN = 0.1
_RMAX = 10.0
_FC_IN = _R * _W * _H
_FC_OUT = 64
_LOG_RMAX_BY_RMIN = math.log(int(_RMAX / float(_RMIN)))

_TILE = 128


def _social_pool_kernel(x_col, y_col, x_row, y_row, hidden, w_fc, b_fc, out):
    xi = x_col[:, 0:1]
    yi = y_col[:, 0:1]
    xj = x_row[0:1, :]
    yj = y_row[0:1, :]

    x_diff = xj - xi
    y_diff = yj - yi
    d2 = x_diff * x_diff + y_diff * y_diff
    r = jnp.sqrt(d2)

    ring_f = jnp.where(
        r < _RMIN,
        jnp.full_like(r, -1.0),
        jnp.floor((_R - 1) * (jnp.log(r / _RMIN) / _LOG_RMAX_BY_RMIN)),
    )
    valid = (ring_f >= 0.0) & (ring_f < _R)
    ring_i = jnp.clip(ring_f, 0.0, _R - 1).astype(jnp.int32)

    theta = jnp.arctan2(y_diff, x_diff)
    wedge = (theta * _W / (2.0 * np.pi) + (_W // 2 - 1)).astype(jnp.int32)
    wedge_m = jnp.mod(wedge, _W)

    cell = ring_i * _W + wedge_m

    h = hidden[...]
    cellb = jnp.where(valid, cell, _R * _W).astype(jnp.bfloat16)
    means = []
    for c in range(_R * _W):
        m = jnp.where(cellb == jnp.bfloat16(c),
                      jnp.bfloat16(1.0), jnp.bfloat16(0.0))
        s = jnp.dot(m, h, preferred_element_type=jnp.float32)
        cnt = s[:, _H:_H + 1]
        sc = s[:, :_H]
        means.append(jnp.where(cnt > 0.0, sc / jnp.maximum(cnt, 1.0), 0.0))

    mean_flat = jnp.concatenate(means, axis=1)
    acc = jnp.dot(mean_flat, w_fc[...], preferred_element_type=jnp.float32,
                  precision=jax.lax.Precision.HIGHEST)
    out[...] = jnp.maximum(acc + b_fc[0:1, :], 0.0)


def kernel(ypred, hidden, W_fc, b_fc):
    yd = jax.lax.stop_gradient(ypred)
    x_col = yd[:, 0:1]
    y_col = yd[:, 1:2]
    x_row = yd[:, 0].reshape(1, _N)
    y_row = yd[:, 1].reshape(1, _N)
    b2 = b_fc.reshape(1, _FC_OUT)
    hidden_aug = jnp.concatenate(
        [hidden, jnp.ones((_N, 1), jnp.float32)], axis=1
    ).astype(jnp.bfloat16)

    grid = (_N // _TILE,)
    return pl.pallas_call(
        _social_pool_kernel,
        grid=grid,
        in_specs=[
            pl.BlockSpec((_TILE, 1), lambda t: (t, 0)),
            pl.BlockSpec((_TILE, 1), lambda t: (t, 0)),
            pl.BlockSpec((1, _N), lambda t: (0, 0)),
            pl.BlockSpec((1, _N), lambda t: (0, 0)),
            pl.BlockSpec((_N, _H + 1), lambda t: (0, 0)),
            pl.BlockSpec((_FC_IN, _FC_OUT), lambda t: (0, 0)),
            pl.BlockSpec((1, _FC_OUT), lambda t: (0, 0)),
        ],
        out_specs=pl.BlockSpec((_TILE, _FC_OUT), lambda t: (t, 0)),
        out_shape=jax.ShapeDtypeStruct((_N, _FC_OUT), jnp.float32),
    )(x_col, y_col, x_row, y_row, hidden_aug, W_fc, b2)


# speedup vs baseline: 77.7953x; 1.1781x over previous
import math

import jax
import jax.numpy as jnp
import numpy as np
from jax.experimental import pallas as pl

_N = 1024
_R = 6
_W = 8
_H = 64
_RMIN = 0.1
_RMAX = 10.0
_FC_IN = _R * _W * _H
_FC_OUT = 64
_LOG_RMAX_BY_RMIN = math.log(int(_RMAX / float(_RMIN)))

_TILE = 128


def _social_pool_kernel(x_col, y_col, x_row, y_row, hidden, w_fc, b_fc, out):
    xi = x_col[:, 0:1]
    yi = y_col[:, 0:1]
    xj = x_row[0:1, :]
    yj = y_row[0:1, :]

    x_diff = xj - xi
    y_diff = yj - yi
    d2 = x_diff * x_diff + y_diff * y_diff
    r = jnp.sqrt(d2)

    ring_f = jnp.where(
        r < _RMIN,
        jnp.full_like(r, -1.0),
        jnp.floor((_R - 1) * (jnp.log(r / _RMIN) / _LOG_RMAX_BY_RMIN)),
    )
    valid = (ring_f >= 0.0) & (ring_f < _R)
    ring_i = jnp.clip(ring_f, 0.0, _R - 1).astype(jnp.int32)

    theta = jnp.arctan2(y_diff, x_diff)
    wedge = (theta * _W / (2.0 * np.pi) + (_W // 2 - 1)).astype(jnp.int32)
    wedge_m = jnp.mod(wedge, _W)

    cell = ring_i * _W + wedge_m

    h = hidden[...]
    cellb = jnp.where(valid, cell, _R * _W).astype(jnp.bfloat16)
    means = []
    for c in range(_R * _W):
        m = jnp.where(cellb == jnp.bfloat16(c),
                      jnp.bfloat16(1.0), jnp.bfloat16(0.0))
        s = jnp.dot(m, h, preferred_element_type=jnp.float32)
        cnt = s[:, _H:_H + 1]
        sc = s[:, :_H]
        rec = 1.0 / jnp.maximum(cnt, 1.0)
        means.append((sc * rec).astype(jnp.bfloat16))

    mean_flat = jnp.concatenate(means, axis=1)
    acc = jnp.dot(mean_flat, w_fc[...], preferred_element_type=jnp.float32)
    out[...] = jnp.maximum(acc + b_fc[0:1, :], 0.0)


def kernel(ypred, hidden, W_fc, b_fc):
    yd = jax.lax.stop_gradient(ypred)
    x_col = yd[:, 0:1]
    y_col = yd[:, 1:2]
    x_row = yd[:, 0].reshape(1, _N)
    y_row = yd[:, 1].reshape(1, _N)
    b2 = b_fc.reshape(1, _FC_OUT)
    hidden_aug = jnp.concatenate(
        [hidden, jnp.ones((_N, 1), jnp.float32)], axis=1
    ).astype(jnp.bfloat16)
    w_bf = W_fc.astype(jnp.bfloat16)

    grid = (_N // _TILE,)
    return pl.pallas_call(
        _social_pool_kernel,
        grid=grid,
        in_specs=[
            pl.BlockSpec((_TILE, 1), lambda t: (t, 0)),
            pl.BlockSpec((_TILE, 1), lambda t: (t, 0)),
            pl.BlockSpec((1, _N), lambda t: (0, 0)),
            pl.BlockSpec((1, _N), lambda t: (0, 0)),
            pl.BlockSpec((_N, _H + 1), lambda t: (0, 0)),
            pl.BlockSpec((_FC_IN, _FC_OUT), lambda t: (0, 0)),
            pl.BlockSpec((1, _FC_OUT), lambda t: (0, 0)),
        ],
        out_specs=pl.BlockSpec((_TILE, _FC_OUT), lambda t: (t, 0)),
        out_shape=jax.ShapeDtypeStruct((_N, _FC_OUT), jnp.float32),
    )(x_col, y_col, x_row, y_row, hidden_aug, w_bf, b2)


# speedup vs baseline: 80.1302x; 1.0300x over previous
import math

import jax
import jax.numpy as jnp
import numpy as np
from jax.experimental import pallas as pl

_N = 1024
_R = 6
_W = 8
_H = 64
_RMIN = 0.1
_RMAX = 10.0
_FC_IN = _R * _W * _H
_FC_OUT = 64
_LOG_RMAX_BY_RMIN = math.log(int(_RMAX / float(_RMIN)))

_TILE = 256


def _social_pool_kernel(x_col, y_col, x_row, y_row, hidden, w_fc, b_fc, out):
    xi = x_col[:, 0:1]
    yi = y_col[:, 0:1]
    xj = x_row[0:1, :]
    yj = y_row[0:1, :]

    x_diff = xj - xi
    y_diff = yj - yi
    d2 = x_diff * x_diff + y_diff * y_diff
    r = jnp.sqrt(d2)

    ring_f = jnp.where(
        r < _RMIN,
        jnp.full_like(r, -1.0),
        jnp.floor((_R - 1) * (jnp.log(r / _RMIN) / _LOG_RMAX_BY_RMIN)),
    )
    valid = (ring_f >= 0.0) & (ring_f < _R)
    ring_i = jnp.clip(ring_f, 0.0, _R - 1).astype(jnp.int32)

    theta = jnp.arctan2(y_diff, x_diff)
    wedge = (theta * _W / (2.0 * np.pi) + (_W // 2 - 1)).astype(jnp.int32)
    wedge_m = jnp.mod(wedge, _W)

    cell = ring_i * _W + wedge_m

    h = hidden[...]
    cellb = jnp.where(valid, cell, _R * _W).astype(jnp.bfloat16)
    means = []
    for c in range(_R * _W):
        m = jnp.where(cellb == jnp.bfloat16(c),
                      jnp.bfloat16(1.0), jnp.bfloat16(0.0))
        s = jnp.dot(m, h, preferred_element_type=jnp.float32)
        cnt = s[:, _H:_H + 1]
        sc = s[:, :_H]
        rec = 1.0 / jnp.maximum(cnt, 1.0)
        means.append((sc * rec).astype(jnp.bfloat16))

    mean_flat = jnp.concatenate(means, axis=1)
    acc = jnp.dot(mean_flat, w_fc[...], preferred_element_type=jnp.float32)
    out[...] = jnp.maximum(acc + b_fc[0:1, :], 0.0)


def kernel(ypred, hidden, W_fc, b_fc):
    yd = jax.lax.stop_gradient(ypred)
    x_col = yd[:, 0:1]
    y_col = yd[:, 1:2]
    x_row = yd[:, 0].reshape(1, _N)
    y_row = yd[:, 1].reshape(1, _N)
    b2 = b_fc.reshape(1, _FC_OUT)
    hidden_aug = jnp.concatenate(
        [hidden, jnp.ones((_N, 1), jnp.float32)], axis=1
    ).astype(jnp.bfloat16)
    w_bf = W_fc.astype(jnp.bfloat16)

    grid = (_N // _TILE,)
    return pl.pallas_call(
        _social_pool_kernel,
        grid=grid,
        in_specs=[
            pl.BlockSpec((_TILE, 1), lambda t: (t, 0)),
            pl.BlockSpec((_TILE, 1), lambda t: (t, 0)),
            pl.BlockSpec((1, _N), lambda t: (0, 0)),
            pl.BlockSpec((1, _N), lambda t: (0, 0)),
            pl.BlockSpec((_N, _H + 1), lambda t: (0, 0)),
            pl.BlockSpec((_FC_IN, _FC_OUT), lambda t: (0, 0)),
            pl.BlockSpec((1, _FC_OUT), lambda t: (0, 0)),
        ],
        out_specs=pl.BlockSpec((_TILE, _FC_OUT), lambda t: (t, 0)),
        out_shape=jax.ShapeDtypeStruct((_N, _FC_OUT), jnp.float32),
    )(x_col, y_col, x_row, y_row, hidden_aug, w_bf, b2)
